# baseline (speedup 1.0000x reference)
import functools

import jax
import jax.numpy as jnp
from jax import lax
from jax.experimental import pallas as pl
from jax.experimental.pallas import tpu as pltpu
from jax.experimental.pallas import tpu_sc as plsc

N_NODES = 10000
NPAD = 10240
NC = 2
NS = 16
NW = NC * NS
CH = 128
ROWS_PER_TILE = NPAD // NS
DEG_W = 16
BM = 1024


RND = 8
SC0_FRAC_NUM, SC0_FRAC_DEN = 3, 4


def _split_chunks(k2):
    k0 = (k2 * SC0_FRAC_NUM // SC0_FRAC_DEN) // RND * RND
    return k0, k2 - k0


def _make_sc_agg(D, k):
    mesh = plsc.VectorSubcoreMesh(core_axis_name="c", subcore_axis_name="s")
    k0, k1 = _split_chunks(2 * k)
    assert k0 % RND == 0 and k1 % RND == 0

    scratch = (
        [pltpu.VMEM((RND, CH), jnp.int32),
         pltpu.VMEM((RND, CH), jnp.int32)]
        + [pltpu.VMEM((CH, D), jnp.float32) for _ in range(RND)]
        + [pltpu.VMEM_SHARED((NPAD, D), jnp.float32)]
        + [pltpu.SemaphoreType.DMA for _ in range(2 * RND)]
    )

    @functools.partial(
        pl.kernel,
        mesh=mesh,
        out_type=jax.ShapeDtypeStruct((NC, NPAD, D), jnp.float32),
        scratch_types=scratch,
        compiler_params=pltpu.CompilerParams(use_tc_tiling_on_sc=False),
    )
    def agg(tab_hbm, src_hbm, dst_hbm, zeros_hbm, out_hbm,
            sidx, didx, *rest):
        bufs = rest[:RND]
        acc = rest[RND]
        gsem = rest[RND + 1:2 * RND + 1]
        ssem = rest[2 * RND + 1:]
        c = lax.axis_index("c")
        s = lax.axis_index("s")
        base = jnp.where(c == 0, s * k0, NS * k0 + s * k1)
        rounds = jnp.where(c == 0, k0 // RND, k1 // RND)
        r0 = s * ROWS_PER_TILE
        pltpu.sync_copy(zeros_hbm.at[pl.ds(r0, ROWS_PER_TILE)],
                        acc.at[pl.ds(r0, ROWS_PER_TILE)])
        plsc.subcore_barrier()

        def round_body(t, carry):
            j0 = base + t * RND
            pltpu.sync_copy(src_hbm.at[pl.ds(j0, RND)], sidx)
            pltpu.sync_copy(dst_hbm.at[pl.ds(j0, RND)], didx)
            gd = [pltpu.async_copy(tab_hbm.at[sidx.at[u]], bufs[u], gsem[u])
                  for u in range(RND)]
            sd = []
            for u in range(RND):
                gd[u].wait()
                sd.append(pltpu.async_copy(bufs[u], acc.at[didx.at[u]],
                                           ssem[u], add=True))
            for u in range(RND):
                sd[u].wait()
            return carry

        lax.fori_loop(0, rounds, round_body, 0)
        plsc.subcore_barrier()
        pltpu.sync_copy(acc.at[pl.ds(r0, ROWS_PER_TILE)],
                        out_hbm.at[c, pl.ds(r0, ROWS_PER_TILE)])

    return agg


def _make_sc_deg(k):
    mesh = plsc.VectorSubcoreMesh(core_axis_name="c", subcore_axis_name="s")
    k0, k1 = _split_chunks(2 * k)

    scratch = [
        pltpu.VMEM((k0, CH), jnp.int32),
        pltpu.VMEM((CH, DEG_W), jnp.float32),
        pltpu.VMEM_SHARED((NPAD, DEG_W), jnp.float32),
        pltpu.SemaphoreType.DMA,
    ]

    @functools.partial(
        pl.kernel,
        mesh=mesh,
        out_type=jax.ShapeDtypeStruct((NC, NPAD, DEG_W), jnp.float32),
        scratch_types=scratch,
        compiler_params=pltpu.CompilerParams(use_tc_tiling_on_sc=False),
    )
    def deg(dst_hbm, zeros_hbm, out_hbm, didx_v, rows_v, acc, sem):
        c = lax.axis_index("c")
        s = lax.axis_index("s")
        base = jnp.where(c == 0, s * k0, NS * k0 + s * k1)
        nch = jnp.where(c == 0, k0, k1)
        r0 = s * ROWS_PER_TILE
        pltpu.sync_copy(zeros_hbm.at[pl.ds(r0, ROWS_PER_TILE)],
                        acc.at[pl.ds(r0, ROWS_PER_TILE)])
        pltpu.sync_copy(dst_hbm.at[pl.ds(base, k0)], didx_v)
        ones = jnp.full((16,), 1.0, jnp.float32)
        for i in range(CH):
            rows_v[i, :] = ones
        plsc.subcore_barrier()

        def fire(j, carry):
            pltpu.async_copy(rows_v, acc.at[didx_v.at[j]], sem, add=True)
            return carry

        def drain(j, carry):
            pltpu.make_async_copy(rows_v, acc.at[didx_v.at[j]], sem).wait()
            return carry

        lax.fori_loop(0, nch, fire, 0)
        lax.fori_loop(0, nch, drain, 0)
        plsc.subcore_barrier()
        pltpu.sync_copy(acc.at[pl.ds(r0, ROWS_PER_TILE)],
                        out_hbm.at[c, pl.ds(r0, ROWS_PER_TILE)])

    return deg


def _tc_matmul(x, w):
    m, kdim = x.shape
    n = w.shape[1]

    def body(x_ref, w_ref, o_ref):
        o_ref[...] = jnp.dot(x_ref[...], w_ref[...],
                             preferred_element_type=jnp.float32)

    return pl.pallas_call(
        body,
        grid=(m // BM,),
        in_specs=[
            pl.BlockSpec((BM, kdim), lambda i: (i, 0)),
            pl.BlockSpec((kdim, n), lambda i: (0, 0)),
        ],
        out_specs=pl.BlockSpec((BM, n), lambda i: (i, 0)),
        out_shape=jax.ShapeDtypeStruct((m, n), jnp.float32),
    )(x, w)


def _tc_scale(h, d0, d1):
    m, n = h.shape
    hh = n // 2

    def body(h_ref, d0_ref, d1_ref, oa_ref, ob_ref):
        dinv = lax.rsqrt(d0_ref[...] + d1_ref[...] + 1.0)
        hs = h_ref[...] * dinv
        oa_ref[...] = hs[:, :hh]
        ob_ref[...] = hs[:, hh:]

    return pl.pallas_call(
        body,
        grid=(m // BM,),
        in_specs=[
            pl.BlockSpec((BM, n), lambda i: (i, 0)),
            pl.BlockSpec((BM, 1), lambda i: (i, 0)),
            pl.BlockSpec((BM, 1), lambda i: (i, 0)),
        ],
        out_specs=[
            pl.BlockSpec((BM, hh), lambda i: (i, 0)),
            pl.BlockSpec((BM, hh), lambda i: (i, 0)),
        ],
        out_shape=[
            jax.ShapeDtypeStruct((m, hh), jnp.float32),
            jax.ShapeDtypeStruct((m, hh), jnp.float32),
        ],
    )(h, d0, d1)


def _tc_mid(pa0, pa1, pb0, pb1, hsa, hsb, d0, d1, b1, w2):
    m, hh = hsa.shape
    n2 = w2.shape[1]

    def body(pa0_ref, pa1_ref, pb0_ref, pb1_ref, hsa_ref, hsb_ref,
             d0_ref, d1_ref, b1_ref, w2_ref, o_ref):
        dinv = lax.rsqrt(d0_ref[...] + d1_ref[...] + 1.0)
        outa = dinv * (pa0_ref[...] + pa1_ref[...] + hsa_ref[...])
        outb = dinv * (pb0_ref[...] + pb1_ref[...] + hsb_ref[...])
        out1 = jnp.concatenate([outa, outb], axis=1) + b1_ref[...]
        a = jnp.maximum(out1, 0.0)
        o_ref[...] = dinv * jnp.dot(a, w2_ref[...],
                                    preferred_element_type=jnp.float32)

    half = pl.BlockSpec((BM, hh), lambda i: (i, 0))
    col = pl.BlockSpec((BM, 1), lambda i: (i, 0))
    return pl.pallas_call(
        body,
        grid=(m // BM,),
        in_specs=[half, half, half, half, half, half, col, col,
                  pl.BlockSpec((1, 2 * hh), lambda i: (0, 0)),
                  pl.BlockSpec((2 * hh, n2), lambda i: (0, 0))],
        out_specs=pl.BlockSpec((BM, n2), lambda i: (i, 0)),
        out_shape=jax.ShapeDtypeStruct((m, n2), jnp.float32),
    )(pa0, pa1, pb0, pb1, hsa, hsb, d0, d1, b1, w2)


def _tc_final(p0, p1, hs2, d0, d1, b2):
    m, n = hs2.shape

    def body(p0_ref, p1_ref, hs2_ref, d0_ref, d1_ref, b2_ref, o_ref):
        dinv = lax.rsqrt(d0_ref[...] + d1_ref[...] + 1.0)
        o_ref[...] = dinv * (p0_ref[...] + p1_ref[...] + hs2_ref[...]) + b2_ref[...]

    return pl.pallas_call(
        body,
        grid=(m // BM,),
        in_specs=[
            pl.BlockSpec((BM, n), lambda i: (i, 0)),
            pl.BlockSpec((BM, n), lambda i: (i, 0)),
            pl.BlockSpec((BM, n), lambda i: (i, 0)),
            pl.BlockSpec((BM, 1), lambda i: (i, 0)),
            pl.BlockSpec((BM, 1), lambda i: (i, 0)),
            pl.BlockSpec((1, n), lambda i: (0, 0)),
        ],
        out_specs=pl.BlockSpec((BM, n), lambda i: (i, 0)),
        out_shape=jax.ShapeDtypeStruct((m, n), jnp.float32),
    )(p0, p1, hs2, d0, d1, b2)


def kernel(x, edge_index, W1, b1, W2, b2):
    n, in_dim = x.shape
    hid = W1.shape[1]
    out_dim = W2.shape[1]
    e = edge_index.shape[1]
    k = -(-e // (NW * CH))
    k += (-k) % RND
    epad = NW * CH * k
    k0, k1 = _split_chunks(2 * k)
    tail = k0 - k1

    src = edge_index[0].astype(jnp.int32)
    dst = edge_index[1].astype(jnp.int32)
    fill = jnp.full((epad - e + tail * CH,), n, jnp.int32)
    src_p = jnp.concatenate([src, fill]).reshape(NW // 2 * (k0 + k1) + tail, CH)
    dst_p = jnp.concatenate([dst, fill]).reshape(NW // 2 * (k0 + k1) + tail, CH)

    x_p = jnp.pad(x, ((0, NPAD - n), (0, 0)))
    b1r = b1.reshape(1, hid)
    b2r = b2.reshape(1, out_dim)

    zeros_w = jnp.zeros((NPAD, DEG_W), jnp.float32)
    zeros_h2 = jnp.zeros((NPAD, hid // 2), jnp.float32)
    zeros_o = jnp.zeros((NPAD, out_dim), jnp.float32)

    pdeg = _make_sc_deg(k)(dst_p, zeros_w)
    h1 = _tc_matmul(x_p, W1)

    d0 = pdeg[0, :, 0:1]
    d1 = pdeg[1, :, 0:1]

    hs1a, hs1b = _tc_scale(h1, d0, d1)
    agg64 = _make_sc_agg(hid // 2, k)
    p1a = agg64(hs1a, src_p, dst_p, zeros_h2)
    p1b = agg64(hs1b, src_p, dst_p, zeros_h2)

    hs2 = _tc_mid(p1a[0], p1a[1], p1b[0], p1b[1], hs1a, hs1b,
                  d0, d1, b1r, W2)
    p2 = _make_sc_agg(out_dim, k)(hs2, src_p, dst_p, zeros_o)

    z = _tc_final(p2[0], p2[1], hs2, d0, d1, b2r)
    return z[:n]


# speedup vs baseline: 1.0029x; 1.0029x over previous
import functools

import jax
import jax.numpy as jnp
from jax import lax
from jax.experimental import pallas as pl
from jax.experimental.pallas import tpu as pltpu
from jax.experimental.pallas import tpu_sc as plsc

N_NODES = 10000
NPAD = 10240
NC = 2
NS = 16
NW = NC * NS
CH = 128
ROWS_PER_TILE = NPAD // NS
DEG_W = 16
BM = 1024


RND = 8
SC0_FRAC_NUM, SC0_FRAC_DEN = 1, 4


def _split_chunks(k2):
    k0 = (k2 * SC0_FRAC_NUM // SC0_FRAC_DEN) // RND * RND
    return k0, k2 - k0


def _make_sc_agg(D, k):
    mesh = plsc.VectorSubcoreMesh(core_axis_name="c", subcore_axis_name="s")
    k0, k1 = _split_chunks(2 * k)
    assert k0 % RND == 0 and k1 % RND == 0

    scratch = (
        [pltpu.VMEM((RND, CH), jnp.int32),
         pltpu.VMEM((RND, CH), jnp.int32)]
        + [pltpu.VMEM((CH, D), jnp.float32) for _ in range(RND)]
        + [pltpu.VMEM_SHARED((NPAD, D), jnp.float32)]
        + [pltpu.SemaphoreType.DMA for _ in range(2 * RND)]
    )

    @functools.partial(
        pl.kernel,
        mesh=mesh,
        out_type=jax.ShapeDtypeStruct((NC, NPAD, D), jnp.float32),
        scratch_types=scratch,
        compiler_params=pltpu.CompilerParams(use_tc_tiling_on_sc=False),
    )
    def agg(tab_hbm, src_hbm, dst_hbm, zeros_hbm, out_hbm,
            sidx, didx, *rest):
        bufs = rest[:RND]
        acc = rest[RND]
        gsem = rest[RND + 1:2 * RND + 1]
        ssem = rest[2 * RND + 1:]
        c = lax.axis_index("c")
        s = lax.axis_index("s")
        base = jnp.where(c == 0, s * k0, NS * k0 + s * k1)
        rounds = jnp.where(c == 0, k0 // RND, k1 // RND)
        r0 = s * ROWS_PER_TILE
        pltpu.sync_copy(zeros_hbm.at[pl.ds(r0, ROWS_PER_TILE)],
                        acc.at[pl.ds(r0, ROWS_PER_TILE)])
        plsc.subcore_barrier()

        def round_body(t, carry):
            j0 = base + t * RND
            pltpu.sync_copy(src_hbm.at[pl.ds(j0, RND)], sidx)
            pltpu.sync_copy(dst_hbm.at[pl.ds(j0, RND)], didx)
            gd = [pltpu.async_copy(tab_hbm.at[sidx.at[u]], bufs[u], gsem[u])
                  for u in range(RND)]
            sd = []
            for u in range(RND):
                gd[u].wait()
                sd.append(pltpu.async_copy(bufs[u], acc.at[didx.at[u]],
                                           ssem[u], add=True))
            for u in range(RND):
                sd[u].wait()
            return carry

        lax.fori_loop(0, rounds, round_body, 0)
        plsc.subcore_barrier()
        pltpu.sync_copy(acc.at[pl.ds(r0, ROWS_PER_TILE)],
                        out_hbm.at[c, pl.ds(r0, ROWS_PER_TILE)])

    return agg


def _make_sc_deg(k):
    mesh = plsc.VectorSubcoreMesh(core_axis_name="c", subcore_axis_name="s")
    k0, k1 = _split_chunks(2 * k)

    kmax = max(k0, k1)
    scratch = [
        pltpu.VMEM((kmax, CH), jnp.int32),
        pltpu.VMEM((CH, DEG_W), jnp.float32),
        pltpu.VMEM_SHARED((NPAD, DEG_W), jnp.float32),
        pltpu.SemaphoreType.DMA,
    ]

    @functools.partial(
        pl.kernel,
        mesh=mesh,
        out_type=jax.ShapeDtypeStruct((NC, NPAD, DEG_W), jnp.float32),
        scratch_types=scratch,
        compiler_params=pltpu.CompilerParams(use_tc_tiling_on_sc=False),
    )
    def deg(dst_hbm, zeros_hbm, out_hbm, didx_v, rows_v, acc, sem):
        c = lax.axis_index("c")
        s = lax.axis_index("s")
        base = jnp.where(c == 0, s * k0, NS * k0 + s * k1)
        nch = jnp.where(c == 0, k0, k1)
        r0 = s * ROWS_PER_TILE
        pltpu.sync_copy(zeros_hbm.at[pl.ds(r0, ROWS_PER_TILE)],
                        acc.at[pl.ds(r0, ROWS_PER_TILE)])
        pltpu.sync_copy(dst_hbm.at[pl.ds(base, kmax)], didx_v)
        ones = jnp.full((16,), 1.0, jnp.float32)
        for i in range(CH):
            rows_v[i, :] = ones
        plsc.subcore_barrier()

        def fire(j, carry):
            pltpu.async_copy(rows_v, acc.at[didx_v.at[j]], sem, add=True)
            return carry

        def drain(j, carry):
            pltpu.make_async_copy(rows_v, acc.at[didx_v.at[j]], sem).wait()
            return carry

        lax.fori_loop(0, nch, fire, 0)
        lax.fori_loop(0, nch, drain, 0)
        plsc.subcore_barrier()
        pltpu.sync_copy(acc.at[pl.ds(r0, ROWS_PER_TILE)],
                        out_hbm.at[c, pl.ds(r0, ROWS_PER_TILE)])

    return deg


def _tc_matmul(x, w):
    m, kdim = x.shape
    n = w.shape[1]

    def body(x_ref, w_ref, o_ref):
        o_ref[...] = jnp.dot(x_ref[...], w_ref[...],
                             preferred_element_type=jnp.float32)

    return pl.pallas_call(
        body,
        grid=(m // BM,),
        in_specs=[
            pl.BlockSpec((BM, kdim), lambda i: (i, 0)),
            pl.BlockSpec((kdim, n), lambda i: (0, 0)),
        ],
        out_specs=pl.BlockSpec((BM, n), lambda i: (i, 0)),
        out_shape=jax.ShapeDtypeStruct((m, n), jnp.float32),
    )(x, w)


def _tc_scale(h, d0, d1):
    m, n = h.shape
    hh = n // 2

    def body(h_ref, d0_ref, d1_ref, oa_ref, ob_ref):
        dinv = lax.rsqrt(d0_ref[...] + d1_ref[...] + 1.0)
        hs = h_ref[...] * dinv
        oa_ref[...] = hs[:, :hh]
        ob_ref[...] = hs[:, hh:]

    return pl.pallas_call(
        body,
        grid=(m // BM,),
        in_specs=[
            pl.BlockSpec((BM, n), lambda i: (i, 0)),
            pl.BlockSpec((BM, 1), lambda i: (i, 0)),
            pl.BlockSpec((BM, 1), lambda i: (i, 0)),
        ],
        out_specs=[
            pl.BlockSpec((BM, hh), lambda i: (i, 0)),
            pl.BlockSpec((BM, hh), lambda i: (i, 0)),
        ],
        out_shape=[
            jax.ShapeDtypeStruct((m, hh), jnp.float32),
            jax.ShapeDtypeStruct((m, hh), jnp.float32),
        ],
    )(h, d0, d1)


def _tc_mid(pa0, pa1, pb0, pb1, hsa, hsb, d0, d1, b1, w2):
    m, hh = hsa.shape
    n2 = w2.shape[1]

    def body(pa0_ref, pa1_ref, pb0_ref, pb1_ref, hsa_ref, hsb_ref,
             d0_ref, d1_ref, b1_ref, w2_ref, o_ref):
        dinv = lax.rsqrt(d0_ref[...] + d1_ref[...] + 1.0)
        outa = dinv * (pa0_ref[...] + pa1_ref[...] + hsa_ref[...])
        outb = dinv * (pb0_ref[...] + pb1_ref[...] + hsb_ref[...])
        out1 = jnp.concatenate([outa, outb], axis=1) + b1_ref[...]
        a = jnp.maximum(out1, 0.0)
        o_ref[...] = dinv * jnp.dot(a, w2_ref[...],
                                    preferred_element_type=jnp.float32)

    half = pl.BlockSpec((BM, hh), lambda i: (i, 0))
    col = pl.BlockSpec((BM, 1), lambda i: (i, 0))
    return pl.pallas_call(
        body,
        grid=(m // BM,),
        in_specs=[half, half, half, half, half, half, col, col,
                  pl.BlockSpec((1, 2 * hh), lambda i: (0, 0)),
                  pl.BlockSpec((2 * hh, n2), lambda i: (0, 0))],
        out_specs=pl.BlockSpec((BM, n2), lambda i: (i, 0)),
        out_shape=jax.ShapeDtypeStruct((m, n2), jnp.float32),
    )(pa0, pa1, pb0, pb1, hsa, hsb, d0, d1, b1, w2)


def _tc_final(p0, p1, hs2, d0, d1, b2):
    m, n = hs2.shape

    def body(p0_ref, p1_ref, hs2_ref, d0_ref, d1_ref, b2_ref, o_ref):
        dinv = lax.rsqrt(d0_ref[...] + d1_ref[...] + 1.0)
        o_ref[...] = dinv * (p0_ref[...] + p1_ref[...] + hs2_ref[...]) + b2_ref[...]

    return pl.pallas_call(
        body,
        grid=(m // BM,),
        in_specs=[
            pl.BlockSpec((BM, n), lambda i: (i, 0)),
            pl.BlockSpec((BM, n), lambda i: (i, 0)),
            pl.BlockSpec((BM, n), lambda i: (i, 0)),
            pl.BlockSpec((BM, 1), lambda i: (i, 0)),
            pl.BlockSpec((BM, 1), lambda i: (i, 0)),
            pl.BlockSpec((1, n), lambda i: (0, 0)),
        ],
        out_specs=pl.BlockSpec((BM, n), lambda i: (i, 0)),
        out_shape=jax.ShapeDtypeStruct((m, n), jnp.float32),
    )(p0, p1, hs2, d0, d1, b2)


def kernel(x, edge_index, W1, b1, W2, b2):
    n, in_dim = x.shape
    hid = W1.shape[1]
    out_dim = W2.shape[1]
    e = edge_index.shape[1]
    k = -(-e // (NW * CH))
    k += (-k) % RND
    epad = NW * CH * k
    k0, k1 = _split_chunks(2 * k)
    tail = max(k0, k1) - k1

    src = edge_index[0].astype(jnp.int32)
    dst = edge_index[1].astype(jnp.int32)
    fill = jnp.full((epad - e + tail * CH,), n, jnp.int32)
    src_p = jnp.concatenate([src, fill]).reshape(NW // 2 * (k0 + k1) + tail, CH)
    dst_p = jnp.concatenate([dst, fill]).reshape(NW // 2 * (k0 + k1) + tail, CH)

    x_p = jnp.pad(x, ((0, NPAD - n), (0, 0)))
    b1r = b1.reshape(1, hid)
    b2r = b2.reshape(1, out_dim)

    zeros_w = jnp.zeros((NPAD, DEG_W), jnp.float32)
    zeros_h2 = jnp.zeros((NPAD, hid // 2), jnp.float32)
    zeros_o = jnp.zeros((NPAD, out_dim), jnp.float32)

    pdeg = _make_sc_deg(k)(dst_p, zeros_w)
    h1 = _tc_matmul(x_p, W1)

    d0 = pdeg[0, :, 0:1]
    d1 = pdeg[1, :, 0:1]

    hs1a, hs1b = _tc_scale(h1, d0, d1)
    agg64 = _make_sc_agg(hid // 2, k)
    p1a = agg64(hs1a, src_p, dst_p, zeros_h2)
    p1b = agg64(hs1b, src_p, dst_p, zeros_h2)

    hs2 = _tc_mid(p1a[0], p1a[1], p1b[0], p1b[1], hs1a, hs1b,
                  d0, d1, b1r, W2)
    p2 = _make_sc_agg(out_dim, k)(hs2, src_p, dst_p, zeros_o)

    z = _tc_final(p2[0], p2[1], hs2, d0, d1, b2r)
    return z[:n]


# speedup vs baseline: 2.1864x; 2.1801x over previous
import functools

import jax
import jax.numpy as jnp
from jax import lax
from jax.experimental import pallas as pl
from jax.experimental.pallas import tpu as pltpu
from jax.experimental.pallas import tpu_sc as plsc

N_NODES = 10000
NPAD = 10240
NC = 2
NS = 16
NW = NC * NS
CH = 128
ROWS_PER_TILE = NPAD // NS
DEG_W = 16
BM = 1024


RND = 4
SC0_FRAC_NUM, SC0_FRAC_DEN = 1, 2


def _split_chunks(k2):
    k0 = (k2 * SC0_FRAC_NUM // SC0_FRAC_DEN) // RND * RND
    return k0, k2 - k0


def _make_sc_agg(D, k):
    mesh = plsc.VectorSubcoreMesh(core_axis_name="c", subcore_axis_name="s")
    k0, k1 = _split_chunks(2 * k)
    assert k0 % RND == 0 and k1 % RND == 0

    scratch = (
        [pltpu.VMEM((RND, CH), jnp.int32),
         pltpu.VMEM((RND, CH), jnp.int32)]
        + [pltpu.VMEM((CH, D), jnp.float32) for _ in range(RND)]
        + [pltpu.VMEM_SHARED((NPAD, D), jnp.float32),
           pltpu.VMEM_SHARED((NPAD, D), jnp.float32)]
        + [pltpu.SemaphoreType.DMA for _ in range(2 * RND)]
    )

    @functools.partial(
        pl.kernel,
        mesh=mesh,
        out_type=jax.ShapeDtypeStruct((NC, NPAD, D), jnp.float32),
        scratch_types=scratch,
        compiler_params=pltpu.CompilerParams(use_tc_tiling_on_sc=False),
    )
    def agg(tab_hbm, src_hbm, dst_hbm, zeros_hbm, out_hbm,
            sidx, didx, *rest):
        bufs = rest[:RND]
        acc = rest[RND]
        tabs = rest[RND + 1]
        gsem = rest[RND + 2:2 * RND + 2]
        ssem = rest[2 * RND + 2:]
        c = lax.axis_index("c")
        s = lax.axis_index("s")
        base = jnp.where(c == 0, s * k0, NS * k0 + s * k1)
        rounds = jnp.where(c == 0, k0 // RND, k1 // RND)
        r0 = s * ROWS_PER_TILE
        pltpu.sync_copy(zeros_hbm.at[pl.ds(r0, ROWS_PER_TILE)],
                        acc.at[pl.ds(r0, ROWS_PER_TILE)])
        pltpu.sync_copy(tab_hbm.at[pl.ds(r0, ROWS_PER_TILE)],
                        tabs.at[pl.ds(r0, ROWS_PER_TILE)])
        plsc.subcore_barrier()

        def round_body(t, carry):
            j0 = base + t * RND
            pltpu.sync_copy(src_hbm.at[pl.ds(j0, RND)], sidx)
            pltpu.sync_copy(dst_hbm.at[pl.ds(j0, RND)], didx)
            gd = [pltpu.async_copy(tabs.at[sidx.at[u]], bufs[u], gsem[u])
                  for u in range(RND)]
            sd = []
            for u in range(RND):
                gd[u].wait()
                sd.append(pltpu.async_copy(bufs[u], acc.at[didx.at[u]],
                                           ssem[u], add=True))
            for u in range(RND):
                sd[u].wait()
            return carry

        lax.fori_loop(0, rounds, round_body, 0)
        plsc.subcore_barrier()
        pltpu.sync_copy(acc.at[pl.ds(r0, ROWS_PER_TILE)],
                        out_hbm.at[c, pl.ds(r0, ROWS_PER_TILE)])

    return agg


def _make_sc_deg(k):
    mesh = plsc.VectorSubcoreMesh(core_axis_name="c", subcore_axis_name="s")
    k0, k1 = _split_chunks(2 * k)

    kmax = max(k0, k1)
    scratch = [
        pltpu.VMEM((kmax, CH), jnp.int32),
        pltpu.VMEM((CH, DEG_W), jnp.float32),
        pltpu.VMEM_SHARED((NPAD, DEG_W), jnp.float32),
        pltpu.SemaphoreType.DMA,
    ]

    @functools.partial(
        pl.kernel,
        mesh=mesh,
        out_type=jax.ShapeDtypeStruct((NC, NPAD, DEG_W), jnp.float32),
        scratch_types=scratch,
        compiler_params=pltpu.CompilerParams(use_tc_tiling_on_sc=False),
    )
    def deg(dst_hbm, zeros_hbm, out_hbm, didx_v, rows_v, acc, sem):
        c = lax.axis_index("c")
        s = lax.axis_index("s")
        base = jnp.where(c == 0, s * k0, NS * k0 + s * k1)
        nch = jnp.where(c == 0, k0, k1)
        r0 = s * ROWS_PER_TILE
        pltpu.sync_copy(zeros_hbm.at[pl.ds(r0, ROWS_PER_TILE)],
                        acc.at[pl.ds(r0, ROWS_PER_TILE)])
        pltpu.sync_copy(dst_hbm.at[pl.ds(base, kmax)], didx_v)
        ones = jnp.full((16,), 1.0, jnp.float32)
        for i in range(CH):
            rows_v[i, :] = ones
        plsc.subcore_barrier()

        def fire(j, carry):
            pltpu.async_copy(rows_v, acc.at[didx_v.at[j]], sem, add=True)
            return carry

        def drain(j, carry):
            pltpu.make_async_copy(rows_v, acc.at[didx_v.at[j]], sem).wait()
            return carry

        lax.fori_loop(0, nch, fire, 0)
        lax.fori_loop(0, nch, drain, 0)
        plsc.subcore_barrier()
        pltpu.sync_copy(acc.at[pl.ds(r0, ROWS_PER_TILE)],
                        out_hbm.at[c, pl.ds(r0, ROWS_PER_TILE)])

    return deg


def _tc_matmul(x, w):
    m, kdim = x.shape
    n = w.shape[1]

    def body(x_ref, w_ref, o_ref):
        o_ref[...] = jnp.dot(x_ref[...], w_ref[...],
                             preferred_element_type=jnp.float32)

    return pl.pallas_call(
        body,
        grid=(m // BM,),
        in_specs=[
            pl.BlockSpec((BM, kdim), lambda i: (i, 0)),
            pl.BlockSpec((kdim, n), lambda i: (0, 0)),
        ],
        out_specs=pl.BlockSpec((BM, n), lambda i: (i, 0)),
        out_shape=jax.ShapeDtypeStruct((m, n), jnp.float32),
    )(x, w)


def _tc_scale(h, d0, d1):
    m, n = h.shape
    hh = n // 2

    def body(h_ref, d0_ref, d1_ref, oa_ref, ob_ref):
        dinv = lax.rsqrt(d0_ref[...] + d1_ref[...] + 1.0)
        hs = h_ref[...] * dinv
        oa_ref[...] = hs[:, :hh]
        ob_ref[...] = hs[:, hh:]

    return pl.pallas_call(
        body,
        grid=(m // BM,),
        in_specs=[
            pl.BlockSpec((BM, n), lambda i: (i, 0)),
            pl.BlockSpec((BM, 1), lambda i: (i, 0)),
            pl.BlockSpec((BM, 1), lambda i: (i, 0)),
        ],
        out_specs=[
            pl.BlockSpec((BM, hh), lambda i: (i, 0)),
            pl.BlockSpec((BM, hh), lambda i: (i, 0)),
        ],
        out_shape=[
            jax.ShapeDtypeStruct((m, hh), jnp.float32),
            jax.ShapeDtypeStruct((m, hh), jnp.float32),
        ],
    )(h, d0, d1)


def _tc_mid(pa0, pa1, pb0, pb1, hsa, hsb, d0, d1, b1, w2):
    m, hh = hsa.shape
    n2 = w2.shape[1]

    def body(pa0_ref, pa1_ref, pb0_ref, pb1_ref, hsa_ref, hsb_ref,
             d0_ref, d1_ref, b1_ref, w2_ref, o_ref):
        dinv = lax.rsqrt(d0_ref[...] + d1_ref[...] + 1.0)
        outa = dinv * (pa0_ref[...] + pa1_ref[...] + hsa_ref[...])
        outb = dinv * (pb0_ref[...] + pb1_ref[...] + hsb_ref[...])
        out1 = jnp.concatenate([outa, outb], axis=1) + b1_ref[...]
        a = jnp.maximum(out1, 0.0)
        o_ref[...] = dinv * jnp.dot(a, w2_ref[...],
                                    preferred_element_type=jnp.float32)

    half = pl.BlockSpec((BM, hh), lambda i: (i, 0))
    col = pl.BlockSpec((BM, 1), lambda i: (i, 0))
    return pl.pallas_call(
        body,
        grid=(m // BM,),
        in_specs=[half, half, half, half, half, half, col, col,
                  pl.BlockSpec((1, 2 * hh), lambda i: (0, 0)),
                  pl.BlockSpec((2 * hh, n2), lambda i: (0, 0))],
        out_specs=pl.BlockSpec((BM, n2), lambda i: (i, 0)),
        out_shape=jax.ShapeDtypeStruct((m, n2), jnp.float32),
    )(pa0, pa1, pb0, pb1, hsa, hsb, d0, d1, b1, w2)


def _tc_final(p0, p1, hs2, d0, d1, b2):
    m, n = hs2.shape

    def body(p0_ref, p1_ref, hs2_ref, d0_ref, d1_ref, b2_ref, o_ref):
        dinv = lax.rsqrt(d0_ref[...] + d1_ref[...] + 1.0)
        o_ref[...] = dinv * (p0_ref[...] + p1_ref[...] + hs2_ref[...]) + b2_ref[...]

    return pl.pallas_call(
        body,
        grid=(m // BM,),
        in_specs=[
            pl.BlockSpec((BM, n), lambda i: (i, 0)),
            pl.BlockSpec((BM, n), lambda i: (i, 0)),
            pl.BlockSpec((BM, n), lambda i: (i, 0)),
            pl.BlockSpec((BM, 1), lambda i: (i, 0)),
            pl.BlockSpec((BM, 1), lambda i: (i, 0)),
            pl.BlockSpec((1, n), lambda i: (0, 0)),
        ],
        out_specs=pl.BlockSpec((BM, n), lambda i: (i, 0)),
        out_shape=jax.ShapeDtypeStruct((m, n), jnp.float32),
    )(p0, p1, hs2, d0, d1, b2)


def kernel(x, edge_index, W1, b1, W2, b2):
    n, in_dim = x.shape
    hid = W1.shape[1]
    out_dim = W2.shape[1]
    e = edge_index.shape[1]
    k = -(-e // (NW * CH))
    k += (-k) % RND
    epad = NW * CH * k
    k0, k1 = _split_chunks(2 * k)
    tail = max(k0, k1) - k1

    src = edge_index[0].astype(jnp.int32)
    dst = edge_index[1].astype(jnp.int32)
    fill = jnp.full((epad - e + tail * CH,), n, jnp.int32)
    src_p = jnp.concatenate([src, fill]).reshape(NW // 2 * (k0 + k1) + tail, CH)
    dst_p = jnp.concatenate([dst, fill]).reshape(NW // 2 * (k0 + k1) + tail, CH)

    x_p = jnp.pad(x, ((0, NPAD - n), (0, 0)))
    b1r = b1.reshape(1, hid)
    b2r = b2.reshape(1, out_dim)

    zeros_w = jnp.zeros((NPAD, DEG_W), jnp.float32)
    zeros_h2 = jnp.zeros((NPAD, hid // 2), jnp.float32)
    zeros_o = jnp.zeros((NPAD, out_dim), jnp.float32)

    pdeg = _make_sc_deg(k)(dst_p, zeros_w)
    h1 = _tc_matmul(x_p, W1)

    d0 = pdeg[0, :, 0:1]
    d1 = pdeg[1, :, 0:1]

    hs1a, hs1b = _tc_scale(h1, d0, d1)
    agg64 = _make_sc_agg(hid // 2, k)
    p1a = agg64(hs1a, src_p, dst_p, zeros_h2)
    p1b = agg64(hs1b, src_p, dst_p, zeros_h2)

    hs2 = _tc_mid(p1a[0], p1a[1], p1b[0], p1b[1], hs1a, hs1b,
                  d0, d1, b1r, W2)
    p2 = _make_sc_agg(out_dim, k)(hs2, src_p, dst_p, zeros_o)

    z = _tc_final(p2[0], p2[1], hs2, d0, d1, b2r)
    return z[:n]


# speedup vs baseline: 2.3370x; 1.0689x over previous
import functools

import jax
import jax.numpy as jnp
from jax import lax
from jax.experimental import pallas as pl
from jax.experimental.pallas import tpu as pltpu
from jax.experimental.pallas import tpu_sc as plsc

N_NODES = 10000
NPAD = 10240
NC = 2
NS = 16
NW = NC * NS
CH = 128
ROWS_PER_TILE = NPAD // NS
DEG_W = 16
BM = 1024


RND = 4
SC0_FRAC_NUM, SC0_FRAC_DEN = 1, 2


def _split_chunks(k2):
    k0 = (k2 * SC0_FRAC_NUM // SC0_FRAC_DEN) // RND * RND
    return k0, k2 - k0


def _make_sc_agg(D, k):
    mesh = plsc.VectorSubcoreMesh(core_axis_name="c", subcore_axis_name="s")
    k0, k1 = _split_chunks(2 * k)
    assert k0 % RND == 0 and k1 % RND == 0

    scratch = (
        [pltpu.VMEM((RND, CH), jnp.int32),
         pltpu.VMEM((RND, CH), jnp.int32)]
        + [pltpu.VMEM((CH, D), jnp.float32) for _ in range(RND)]
        + [pltpu.VMEM_SHARED((NPAD, D), jnp.float32),
           pltpu.VMEM_SHARED((NPAD, D), jnp.float32)]
        + [pltpu.SemaphoreType.DMA for _ in range(2 * RND)]
    )

    @functools.partial(
        pl.kernel,
        mesh=mesh,
        out_type=jax.ShapeDtypeStruct((NC, NPAD, D), jnp.float32),
        scratch_types=scratch,
        compiler_params=pltpu.CompilerParams(use_tc_tiling_on_sc=False),
    )
    def agg(tab_hbm, src_hbm, dst_hbm, zeros_hbm, out_hbm,
            sidx, didx, *rest):
        bufs = rest[:RND]
        acc = rest[RND]
        tabs = rest[RND + 1]
        gsem = rest[RND + 2:2 * RND + 2]
        ssem = rest[2 * RND + 2:]
        c = lax.axis_index("c")
        s = lax.axis_index("s")
        base = jnp.where(c == 0, s * k0, NS * k0 + s * k1)
        rounds = jnp.where(c == 0, k0 // RND, k1 // RND)
        r0 = s * ROWS_PER_TILE
        pltpu.sync_copy(zeros_hbm.at[pl.ds(r0, ROWS_PER_TILE)],
                        acc.at[pl.ds(r0, ROWS_PER_TILE)])
        pltpu.sync_copy(tab_hbm.at[pl.ds(r0, ROWS_PER_TILE)],
                        tabs.at[pl.ds(r0, ROWS_PER_TILE)])
        plsc.subcore_barrier()

        def round_body(t, carry):
            j0 = base + t * RND
            pltpu.sync_copy(src_hbm.at[pl.ds(j0, RND)], sidx)
            pltpu.sync_copy(dst_hbm.at[pl.ds(j0, RND)], didx)
            gd = [pltpu.async_copy(tabs.at[sidx.at[u]], bufs[u], gsem[u])
                  for u in range(RND)]
            sd = []
            for u in range(RND):
                gd[u].wait()
                sd.append(pltpu.async_copy(bufs[u], acc.at[didx.at[u]],
                                           ssem[u], add=True))
            for u in range(RND):
                sd[u].wait()
            return carry

        lax.fori_loop(0, rounds, round_body, 0)
        plsc.subcore_barrier()
        pltpu.sync_copy(acc.at[pl.ds(r0, ROWS_PER_TILE)],
                        out_hbm.at[c, pl.ds(r0, ROWS_PER_TILE)])

    return agg


def _make_sc_deg(k):
    mesh = plsc.VectorSubcoreMesh(core_axis_name="c", subcore_axis_name="s")
    k0, k1 = _split_chunks(2 * k)

    kmax = max(k0, k1)
    scratch = [
        pltpu.VMEM((kmax, CH), jnp.int32),
        pltpu.VMEM((CH, DEG_W), jnp.float32),
        pltpu.VMEM_SHARED((NPAD, DEG_W), jnp.float32),
        pltpu.SemaphoreType.DMA,
    ]

    @functools.partial(
        pl.kernel,
        mesh=mesh,
        out_type=jax.ShapeDtypeStruct((NC, NPAD, DEG_W), jnp.float32),
        scratch_types=scratch,
        compiler_params=pltpu.CompilerParams(use_tc_tiling_on_sc=False),
    )
    def deg(dst_hbm, zeros_hbm, out_hbm, didx_v, rows_v, acc, sem):
        c = lax.axis_index("c")
        s = lax.axis_index("s")
        base = jnp.where(c == 0, s * k0, NS * k0 + s * k1)
        nch = jnp.where(c == 0, k0, k1)
        r0 = s * ROWS_PER_TILE
        pltpu.sync_copy(zeros_hbm.at[pl.ds(r0, ROWS_PER_TILE)],
                        acc.at[pl.ds(r0, ROWS_PER_TILE)])
        pltpu.sync_copy(dst_hbm.at[pl.ds(base, kmax)], didx_v)
        ones = jnp.full((16,), 1.0, jnp.float32)
        for i in range(CH):
            rows_v[i, :] = ones
        plsc.subcore_barrier()

        def fire(j, carry):
            pltpu.async_copy(rows_v, acc.at[didx_v.at[j]], sem, add=True)
            return carry

        def drain(j, carry):
            pltpu.make_async_copy(rows_v, acc.at[didx_v.at[j]], sem).wait()
            return carry

        lax.fori_loop(0, nch, fire, 0)
        lax.fori_loop(0, nch, drain, 0)
        plsc.subcore_barrier()
        pltpu.sync_copy(acc.at[pl.ds(r0, ROWS_PER_TILE)],
                        out_hbm.at[c, pl.ds(r0, ROWS_PER_TILE)])

    return deg


def _tc_matmul(x, w):
    m, kdim = x.shape
    n = w.shape[1]

    def body(x_ref, w_ref, o_ref):
        o_ref[...] = jnp.dot(x_ref[...], w_ref[...],
                             preferred_element_type=jnp.float32)

    return pl.pallas_call(
        body,
        grid=(m // BM,),
        in_specs=[
            pl.BlockSpec((BM, kdim), lambda i: (i, 0)),
            pl.BlockSpec((kdim, n), lambda i: (0, 0)),
        ],
        out_specs=pl.BlockSpec((BM, n), lambda i: (i, 0)),
        out_shape=jax.ShapeDtypeStruct((m, n), jnp.float32),
    )(x, w)


def _dinv_of(pd_blk):
    return lax.rsqrt(pd_blk[0, :, 0:1] + pd_blk[1, :, 0:1] + 1.0)


def _tc_scale(h, pdeg):
    m, n = h.shape
    hh = n // 2

    def body(h_ref, pd_ref, oa_ref, ob_ref):
        dinv = _dinv_of(pd_ref[...])
        hs = h_ref[...] * dinv
        oa_ref[...] = hs[:, :hh]
        ob_ref[...] = hs[:, hh:]

    return pl.pallas_call(
        body,
        grid=(m // BM,),
        in_specs=[
            pl.BlockSpec((BM, n), lambda i: (i, 0)),
            pl.BlockSpec((2, BM, DEG_W), lambda i: (0, i, 0)),
        ],
        out_specs=[
            pl.BlockSpec((BM, hh), lambda i: (i, 0)),
            pl.BlockSpec((BM, hh), lambda i: (i, 0)),
        ],
        out_shape=[
            jax.ShapeDtypeStruct((m, hh), jnp.float32),
            jax.ShapeDtypeStruct((m, hh), jnp.float32),
        ],
    )(h, pdeg)


def _tc_mid(pa, pb, hsa, hsb, pdeg, b1, w2):
    m, hh = hsa.shape
    n2 = w2.shape[1]

    def body(pa_ref, pb_ref, hsa_ref, hsb_ref, pd_ref, b1_ref, w2_ref,
             o_ref):
        dinv = _dinv_of(pd_ref[...])
        outa = dinv * (pa_ref[0] + pa_ref[1] + hsa_ref[...])
        outb = dinv * (pb_ref[0] + pb_ref[1] + hsb_ref[...])
        out1 = jnp.concatenate([outa, outb], axis=1) + b1_ref[...]
        a = jnp.maximum(out1, 0.0)
        o_ref[...] = dinv * jnp.dot(a, w2_ref[...],
                                    preferred_element_type=jnp.float32)

    part = pl.BlockSpec((2, BM, hh), lambda i: (0, i, 0))
    half = pl.BlockSpec((BM, hh), lambda i: (i, 0))
    return pl.pallas_call(
        body,
        grid=(m // BM,),
        in_specs=[part, part, half, half,
                  pl.BlockSpec((2, BM, DEG_W), lambda i: (0, i, 0)),
                  pl.BlockSpec((1, 2 * hh), lambda i: (0, 0)),
                  pl.BlockSpec((2 * hh, n2), lambda i: (0, 0))],
        out_specs=pl.BlockSpec((BM, n2), lambda i: (i, 0)),
        out_shape=jax.ShapeDtypeStruct((m, n2), jnp.float32),
    )(pa, pb, hsa, hsb, pdeg, b1, w2)


def _tc_final(p2, hs2, pdeg, b2, n_out):
    m, n = hs2.shape
    bm = 1000
    assert n_out % bm == 0

    def body(p_ref, hs2_ref, pd_ref, b2_ref, o_ref):
        dinv = _dinv_of(pd_ref[...])
        o_ref[...] = dinv * (p_ref[0] + p_ref[1] + hs2_ref[...]) + b2_ref[...]

    return pl.pallas_call(
        body,
        grid=(n_out // bm,),
        in_specs=[
            pl.BlockSpec((2, bm, n), lambda i: (0, i, 0)),
            pl.BlockSpec((bm, n), lambda i: (i, 0)),
            pl.BlockSpec((2, bm, DEG_W), lambda i: (0, i, 0)),
            pl.BlockSpec((1, n), lambda i: (0, 0)),
        ],
        out_specs=pl.BlockSpec((bm, n), lambda i: (i, 0)),
        out_shape=jax.ShapeDtypeStruct((n_out, n), jnp.float32),
    )(p2, hs2, pdeg, b2)


def kernel(x, edge_index, W1, b1, W2, b2):
    n, in_dim = x.shape
    hid = W1.shape[1]
    out_dim = W2.shape[1]
    e = edge_index.shape[1]
    k = -(-e // (NW * CH))
    k += (-k) % RND
    epad = NW * CH * k
    k0, k1 = _split_chunks(2 * k)
    tail = max(k0, k1) - k1

    src = edge_index[0].astype(jnp.int32)
    dst = edge_index[1].astype(jnp.int32)
    fill = jnp.full((epad - e + tail * CH,), n, jnp.int32)
    src_p = jnp.concatenate([src, fill]).reshape(NW // 2 * (k0 + k1) + tail, CH)
    dst_p = jnp.concatenate([dst, fill]).reshape(NW // 2 * (k0 + k1) + tail, CH)

    x_p = jnp.pad(x, ((0, NPAD - n), (0, 0)))
    b1r = b1.reshape(1, hid)
    b2r = b2.reshape(1, out_dim)

    zeros_w = jnp.zeros((NPAD, DEG_W), jnp.float32)
    zeros_h2 = jnp.zeros((NPAD, hid // 2), jnp.float32)
    zeros_o = jnp.zeros((NPAD, out_dim), jnp.float32)

    pdeg = _make_sc_deg(k)(dst_p, zeros_w)
    h1 = _tc_matmul(x_p, W1)

    hs1a, hs1b = _tc_scale(h1, pdeg)
    agg64 = _make_sc_agg(hid // 2, k)
    p1a = agg64(hs1a, src_p, dst_p, zeros_h2)
    p1b = agg64(hs1b, src_p, dst_p, zeros_h2)

    hs2 = _tc_mid(p1a, p1b, hs1a, hs1b, pdeg, b1r, W2)
    p2 = _make_sc_agg(out_dim, k)(hs2, src_p, dst_p, zeros_o)

    return _tc_final(p2, hs2, pdeg, b2r, n)


# speedup vs baseline: 2.4373x; 1.0429x over previous
import functools

import jax
import jax.numpy as jnp
from jax import lax
from jax.experimental import pallas as pl
from jax.experimental.pallas import tpu as pltpu
from jax.experimental.pallas import tpu_sc as plsc

N_NODES = 10000
NPAD = 10240
NC = 2
NS = 16
NW = NC * NS
CH = 128
ROWS_PER_TILE = NPAD // NS
DEG_W = 16
BM = 1024


RND = 4
SC0_FRAC_NUM, SC0_FRAC_DEN = 1, 2


def _split_chunks(k2):
    k0 = (k2 * SC0_FRAC_NUM // SC0_FRAC_DEN) // RND * RND
    return k0, k2 - k0


def _make_sc_agg(D, k):
    mesh = plsc.VectorSubcoreMesh(core_axis_name="c", subcore_axis_name="s")
    k0, k1 = _split_chunks(2 * k)
    assert k0 % RND == 0 and k1 % RND == 0

    scratch = (
        [pltpu.VMEM((RND, CH), jnp.int32),
         pltpu.VMEM((RND, CH), jnp.int32)]
        + [pltpu.VMEM((CH, D), jnp.float32) for _ in range(RND)]
        + [pltpu.VMEM_SHARED((NPAD, D), jnp.float32),
           pltpu.VMEM_SHARED((NPAD, D), jnp.float32)]
        + [pltpu.SemaphoreType.DMA for _ in range(2 * RND)]
    )

    @functools.partial(
        pl.kernel,
        mesh=mesh,
        out_type=jax.ShapeDtypeStruct((NC, NPAD, D), jnp.float32),
        scratch_types=scratch,
        compiler_params=pltpu.CompilerParams(use_tc_tiling_on_sc=False),
    )
    def agg(tab_hbm, src_hbm, dst_hbm, zeros_hbm, out_hbm,
            sidx, didx, *rest):
        bufs = rest[:RND]
        acc = rest[RND]
        tabs = rest[RND + 1]
        gsem = rest[RND + 2:2 * RND + 2]
        ssem = rest[2 * RND + 2:]
        c = lax.axis_index("c")
        s = lax.axis_index("s")
        base = jnp.where(c == 0, s * k0, NS * k0 + s * k1)
        rounds = jnp.where(c == 0, k0 // RND, k1 // RND)
        r0 = s * ROWS_PER_TILE
        pltpu.sync_copy(zeros_hbm.at[pl.ds(r0, ROWS_PER_TILE)],
                        acc.at[pl.ds(r0, ROWS_PER_TILE)])
        pltpu.sync_copy(tab_hbm.at[pl.ds(r0, ROWS_PER_TILE)],
                        tabs.at[pl.ds(r0, ROWS_PER_TILE)])
        plsc.subcore_barrier()

        def round_body(t, carry):
            j0 = base + t * RND
            pltpu.sync_copy(src_hbm.at[pl.ds(j0, RND)], sidx)
            pltpu.sync_copy(dst_hbm.at[pl.ds(j0, RND)], didx)
            gd = [pltpu.async_copy(tabs.at[sidx.at[u]], bufs[u], gsem[u])
                  for u in range(RND)]
            sd = []
            for u in range(RND):
                gd[u].wait()
                sd.append(pltpu.async_copy(bufs[u], acc.at[didx.at[u]],
                                           ssem[u], add=True))
            for u in range(RND):
                sd[u].wait()
            return carry

        lax.fori_loop(0, rounds, round_body, 0)
        plsc.subcore_barrier()
        pltpu.sync_copy(acc.at[pl.ds(r0, ROWS_PER_TILE)],
                        out_hbm.at[c, pl.ds(r0, ROWS_PER_TILE)])

    return agg


def _make_sc_agg2(D, k2):
    mesh = plsc.VectorSubcoreMesh(core_axis_name="c", subcore_axis_name="s")
    assert k2 % RND == 0

    scratch = (
        [pltpu.VMEM((RND, CH), jnp.int32),
         pltpu.VMEM((RND, CH), jnp.int32)]
        + [pltpu.VMEM((CH, D), jnp.float32) for _ in range(RND)]
        + [pltpu.VMEM_SHARED((NPAD, D), jnp.float32),
           pltpu.VMEM_SHARED((NPAD, D), jnp.float32)]
        + [pltpu.SemaphoreType.DMA for _ in range(2 * RND)]
    )

    @functools.partial(
        pl.kernel,
        mesh=mesh,
        out_type=jax.ShapeDtypeStruct((NC, NPAD, D), jnp.float32),
        scratch_types=scratch,
        compiler_params=pltpu.CompilerParams(use_tc_tiling_on_sc=False),
    )
    def agg2(taba_hbm, tabb_hbm, src_hbm, dst_hbm, zeros_hbm, out_hbm,
             sidx, didx, *rest):
        bufs = rest[:RND]
        acc = rest[RND]
        tabs = rest[RND + 1]
        gsem = rest[RND + 2:2 * RND + 2]
        ssem = rest[2 * RND + 2:]
        c = lax.axis_index("c")
        s = lax.axis_index("s")
        base = s * k2
        r0 = s * ROWS_PER_TILE
        pltpu.sync_copy(zeros_hbm.at[pl.ds(r0, ROWS_PER_TILE)],
                        acc.at[pl.ds(r0, ROWS_PER_TILE)])

        @pl.when(c == 0)
        def _():
            pltpu.sync_copy(taba_hbm.at[pl.ds(r0, ROWS_PER_TILE)],
                            tabs.at[pl.ds(r0, ROWS_PER_TILE)])

        @pl.when(c == 1)
        def _():
            pltpu.sync_copy(tabb_hbm.at[pl.ds(r0, ROWS_PER_TILE)],
                            tabs.at[pl.ds(r0, ROWS_PER_TILE)])

        plsc.subcore_barrier()

        def round_body(t, carry):
            j0 = base + t * RND
            pltpu.sync_copy(src_hbm.at[pl.ds(j0, RND)], sidx)
            pltpu.sync_copy(dst_hbm.at[pl.ds(j0, RND)], didx)
            gd = [pltpu.async_copy(tabs.at[sidx.at[u]], bufs[u], gsem[u])
                  for u in range(RND)]
            sd = []
            for u in range(RND):
                gd[u].wait()
                sd.append(pltpu.async_copy(bufs[u], acc.at[didx.at[u]],
                                           ssem[u], add=True))
            for u in range(RND):
                sd[u].wait()
            return carry

        lax.fori_loop(0, k2 // RND, round_body, 0)
        plsc.subcore_barrier()
        pltpu.sync_copy(acc.at[pl.ds(r0, ROWS_PER_TILE)],
                        out_hbm.at[c, pl.ds(r0, ROWS_PER_TILE)])

    return agg2


def _make_sc_deg(k):
    mesh = plsc.VectorSubcoreMesh(core_axis_name="c", subcore_axis_name="s")
    k0, k1 = _split_chunks(2 * k)

    kmax = max(k0, k1)
    scratch = [
        pltpu.VMEM((kmax, CH), jnp.int32),
        pltpu.VMEM((CH, DEG_W), jnp.float32),
        pltpu.VMEM_SHARED((NPAD, DEG_W), jnp.float32),
        pltpu.SemaphoreType.DMA,
    ]

    @functools.partial(
        pl.kernel,
        mesh=mesh,
        out_type=jax.ShapeDtypeStruct((NC, NPAD, DEG_W), jnp.float32),
        scratch_types=scratch,
        compiler_params=pltpu.CompilerParams(use_tc_tiling_on_sc=False),
    )
    def deg(dst_hbm, zeros_hbm, out_hbm, didx_v, rows_v, acc, sem):
        c = lax.axis_index("c")
        s = lax.axis_index("s")
        base = jnp.where(c == 0, s * k0, NS * k0 + s * k1)
        nch = jnp.where(c == 0, k0, k1)
        r0 = s * ROWS_PER_TILE
        pltpu.sync_copy(zeros_hbm.at[pl.ds(r0, ROWS_PER_TILE)],
                        acc.at[pl.ds(r0, ROWS_PER_TILE)])
        pltpu.sync_copy(dst_hbm.at[pl.ds(base, kmax)], didx_v)
        ones = jnp.full((16,), 1.0, jnp.float32)
        for i in range(CH):
            rows_v[i, :] = ones
        plsc.subcore_barrier()

        def fire(j, carry):
            pltpu.async_copy(rows_v, acc.at[didx_v.at[j]], sem, add=True)
            return carry

        def drain(j, carry):
            pltpu.make_async_copy(rows_v, acc.at[didx_v.at[j]], sem).wait()
            return carry

        lax.fori_loop(0, nch, fire, 0)
        lax.fori_loop(0, nch, drain, 0)
        plsc.subcore_barrier()
        pltpu.sync_copy(acc.at[pl.ds(r0, ROWS_PER_TILE)],
                        out_hbm.at[c, pl.ds(r0, ROWS_PER_TILE)])

    return deg


def _tc_matmul(x, w):
    m, kdim = x.shape
    n = w.shape[1]

    def body(x_ref, w_ref, o_ref):
        o_ref[...] = jnp.dot(x_ref[...], w_ref[...],
                             preferred_element_type=jnp.float32)

    return pl.pallas_call(
        body,
        grid=(m // BM,),
        in_specs=[
            pl.BlockSpec((BM, kdim), lambda i: (i, 0)),
            pl.BlockSpec((kdim, n), lambda i: (0, 0)),
        ],
        out_specs=pl.BlockSpec((BM, n), lambda i: (i, 0)),
        out_shape=jax.ShapeDtypeStruct((m, n), jnp.float32),
    )(x, w)


def _dinv_of(pd_blk):
    return lax.rsqrt(pd_blk[0, :, 0:1] + pd_blk[1, :, 0:1] + 1.0)


def _tc_scale(h, pdeg):
    m, n = h.shape
    hh = n // 2

    def body(h_ref, pd_ref, oa_ref, ob_ref):
        dinv = _dinv_of(pd_ref[...])
        hs = h_ref[...] * dinv
        oa_ref[...] = hs[:, :hh]
        ob_ref[...] = hs[:, hh:]

    return pl.pallas_call(
        body,
        grid=(m // BM,),
        in_specs=[
            pl.BlockSpec((BM, n), lambda i: (i, 0)),
            pl.BlockSpec((2, BM, DEG_W), lambda i: (0, i, 0)),
        ],
        out_specs=[
            pl.BlockSpec((BM, hh), lambda i: (i, 0)),
            pl.BlockSpec((BM, hh), lambda i: (i, 0)),
        ],
        out_shape=[
            jax.ShapeDtypeStruct((m, hh), jnp.float32),
            jax.ShapeDtypeStruct((m, hh), jnp.float32),
        ],
    )(h, pdeg)


def _tc_mid(p1, hsa, hsb, pdeg, b1, w2):
    m, hh = hsa.shape
    n2 = w2.shape[1]

    def body(p_ref, hsa_ref, hsb_ref, pd_ref, b1_ref, w2_ref, o_ref):
        dinv = _dinv_of(pd_ref[...])
        outa = dinv * (p_ref[0] + hsa_ref[...])
        outb = dinv * (p_ref[1] + hsb_ref[...])
        out1 = jnp.concatenate([outa, outb], axis=1) + b1_ref[...]
        a = jnp.maximum(out1, 0.0)
        o_ref[...] = dinv * jnp.dot(a, w2_ref[...],
                                    preferred_element_type=jnp.float32)

    part = pl.BlockSpec((2, BM, hh), lambda i: (0, i, 0))
    half = pl.BlockSpec((BM, hh), lambda i: (i, 0))
    return pl.pallas_call(
        body,
        grid=(m // BM,),
        in_specs=[part, half, half,
                  pl.BlockSpec((2, BM, DEG_W), lambda i: (0, i, 0)),
                  pl.BlockSpec((1, 2 * hh), lambda i: (0, 0)),
                  pl.BlockSpec((2 * hh, n2), lambda i: (0, 0))],
        out_specs=pl.BlockSpec((BM, n2), lambda i: (i, 0)),
        out_shape=jax.ShapeDtypeStruct((m, n2), jnp.float32),
    )(p1, hsa, hsb, pdeg, b1, w2)


def _tc_final(p2, hs2, pdeg, b2, n_out):
    m, n = hs2.shape
    bm = 1000
    assert n_out % bm == 0

    def body(p_ref, hs2_ref, pd_ref, b2_ref, o_ref):
        dinv = _dinv_of(pd_ref[...])
        o_ref[...] = dinv * (p_ref[0] + p_ref[1] + hs2_ref[...]) + b2_ref[...]

    return pl.pallas_call(
        body,
        grid=(n_out // bm,),
        in_specs=[
            pl.BlockSpec((2, bm, n), lambda i: (0, i, 0)),
            pl.BlockSpec((bm, n), lambda i: (i, 0)),
            pl.BlockSpec((2, bm, DEG_W), lambda i: (0, i, 0)),
            pl.BlockSpec((1, n), lambda i: (0, 0)),
        ],
        out_specs=pl.BlockSpec((bm, n), lambda i: (i, 0)),
        out_shape=jax.ShapeDtypeStruct((n_out, n), jnp.float32),
    )(p2, hs2, pdeg, b2)


def kernel(x, edge_index, W1, b1, W2, b2):
    n, in_dim = x.shape
    hid = W1.shape[1]
    out_dim = W2.shape[1]
    e = edge_index.shape[1]
    k = -(-e // (NW * CH))
    k += (-k) % RND
    epad = NW * CH * k
    k0, k1 = _split_chunks(2 * k)
    tail = max(k0, k1) - k1

    src = edge_index[0].astype(jnp.int32)
    dst = edge_index[1].astype(jnp.int32)
    fill = jnp.full((epad - e + tail * CH,), n, jnp.int32)
    src_p = jnp.concatenate([src, fill]).reshape(NW // 2 * (k0 + k1) + tail, CH)
    dst_p = jnp.concatenate([dst, fill]).reshape(NW // 2 * (k0 + k1) + tail, CH)

    x_p = jnp.pad(x, ((0, NPAD - n), (0, 0)))
    b1r = b1.reshape(1, hid)
    b2r = b2.reshape(1, out_dim)

    zeros_w = jnp.zeros((NPAD, DEG_W), jnp.float32)
    zeros_h2 = jnp.zeros((NPAD, hid // 2), jnp.float32)
    zeros_o = jnp.zeros((NPAD, out_dim), jnp.float32)

    pdeg = _make_sc_deg(k)(dst_p, zeros_w)
    h1 = _tc_matmul(x_p, W1)

    hs1a, hs1b = _tc_scale(h1, pdeg)
    p1 = _make_sc_agg2(hid // 2, 2 * k)(hs1a, hs1b, src_p, dst_p, zeros_h2)

    hs2 = _tc_mid(p1, hs1a, hs1b, pdeg, b1r, W2)
    p2 = _make_sc_agg(out_dim, k)(hs2, src_p, dst_p, zeros_o)

    return _tc_final(p2, hs2, pdeg, b2r, n)
